# Initial kernel scaffold; baseline (speedup 1.0000x reference)
import functools

import jax
import jax.numpy as jnp
from jax import lax
from jax.experimental import pallas as pl
from jax.experimental.pallas import tpu as pltpu
from jax.experimental.pallas import tpu_sc as plsc

N = 50000
E = 1600000
IN_DIM = 18
HID = 16

NC = 2
NS = 16
NW = NC * NS
K = 128

NPAD = 50176
RPT = NPAD // NS
ZR = RPT // 8
CHUNKS = -(-E // (NW * K))
EPAD = NW * K * CHUNKS
EPT = EPAD // NW

_f32 = jnp.float32


def _dense_in_body(x_ref, w_ref, asr_ref, adr_ref, h_ref, sa_ref, sd_ref, b_ref):
    h = jnp.dot(x_ref[...], w_ref[...], preferred_element_type=_f32)
    sa = jnp.dot(h, asr_ref[...], preferred_element_type=_f32)
    sd = jnp.dot(h, adr_ref[...], preferred_element_type=_f32)
    m = jnp.max(sa) + jnp.max(sd)
    h_ref[...] = h
    sa_ref[...] = sa
    sd_ref[...] = sd
    b_ref[0, 0] = jnp.where(m > 0, m, 0.2 * m)


def _dense_in(x, w, a_src, a_dst, d_in):
    return pl.pallas_call(
        _dense_in_body,
        out_shape=[
            jax.ShapeDtypeStruct((NPAD, HID), _f32),
            jax.ShapeDtypeStruct((NPAD, 1), _f32),
            jax.ShapeDtypeStruct((NPAD, 1), _f32),
            jax.ShapeDtypeStruct((1, 1), _f32),
        ],
    )(x, w.reshape(d_in, HID), a_src.reshape(HID, 1), a_dst.reshape(HID, 1))


def _normalize(num_ref, den_ref, h_ref, sa_ref, sd_ref, b_ref, bias_ref):
    sa = sa_ref[...]
    sd = sd_ref[...]
    e = sa + sd
    e = jnp.where(e > 0, e, 0.2 * e)
    ps = jnp.exp(e - b_ref[0, 0])
    num = num_ref[0] + num_ref[1] + ps * h_ref[...]
    den = den_ref[0] + den_ref[1] + ps + 1e-16
    return num / den + bias_ref[...]


def _mid_body(num_ref, den_ref, h_ref, sa_ref, sd_ref, b_ref, bias_ref,
              w_ref, asr_ref, adr_ref,
              h2_ref, sa2_ref, sd2_ref, b2_ref):
    o = _normalize(num_ref, den_ref, h_ref, sa_ref, sd_ref, b_ref, bias_ref)
    o = jnp.where(o > 0, o, jnp.expm1(o))
    h2 = jnp.dot(o, w_ref[...], preferred_element_type=_f32)
    sa2 = jnp.dot(h2, asr_ref[...], preferred_element_type=_f32)
    sd2 = jnp.dot(h2, adr_ref[...], preferred_element_type=_f32)
    m = jnp.max(sa2) + jnp.max(sd2)
    h2_ref[...] = h2
    sa2_ref[...] = sa2
    sd2_ref[...] = sd2
    b2_ref[0, 0] = jnp.where(m > 0, m, 0.2 * m)


def _mid(num, den, h, sa, sd, b, bias, w, a_src, a_dst):
    return pl.pallas_call(
        _mid_body,
        out_shape=[
            jax.ShapeDtypeStruct((NPAD, HID), _f32),
            jax.ShapeDtypeStruct((NPAD, 1), _f32),
            jax.ShapeDtypeStruct((NPAD, 1), _f32),
            jax.ShapeDtypeStruct((1, 1), _f32),
        ],
    )(num, den, h, sa, sd, b, bias.reshape(1, HID),
      w.reshape(HID, HID), a_src.reshape(HID, 1), a_dst.reshape(HID, 1))


def _final_body(num_ref, den_ref, h_ref, sa_ref, sd_ref, b_ref, bias_ref, o_ref):
    o_ref[...] = _normalize(num_ref, den_ref, h_ref, sa_ref, sd_ref, b_ref,
                            bias_ref)


def _final(num, den, h, sa, sd, b, bias):
    return pl.pallas_call(
        _final_body,
        out_shape=jax.ShapeDtypeStruct((NPAD, HID), _f32),
    )(num, den, h, sa, sd, b, bias.reshape(1, HID))


@functools.partial(
    pl.kernel,
    out_type=[
        jax.ShapeDtypeStruct((NC, NPAD, HID), _f32),
        jax.ShapeDtypeStruct((NC, NPAD), _f32),
    ],
    mesh=plsc.VectorSubcoreMesh(core_axis_name="c", subcore_axis_name="s"),
    scratch_types=[
        pltpu.VMEM((NPAD,), _f32),
        pltpu.VMEM((NPAD,), _f32),
        pltpu.VMEM((16,), _f32),
        pltpu.VMEM((K,), jnp.int32),
        pltpu.VMEM((K,), jnp.int32),
        pltpu.VMEM((K, HID), _f32),
        pltpu.VMEM((K,), _f32),
        pltpu.VMEM((ZR, HID), _f32),
        pltpu.VMEM((RPT,), _f32),
        pltpu.VMEM_SHARED((NPAD, HID), _f32),
        pltpu.VMEM_SHARED((NPAD,), _f32),
        pltpu.SemaphoreType.DMA,
    ],
)
def _sc_edge_pass(src_hbm, dst_hbm, h_hbm, sa_hbm, sd_hbm, b_hbm,
                  num_out, den_out,
                  sa_t, sd_t, b_v, src_v, dst_v, rows_v, p_v, znum, zden,
                  num_sh, den_sh, sem):
    c = lax.axis_index("c")
    s = lax.axis_index("s")
    wid = c * NS + s

    pltpu.sync_copy(sa_hbm, sa_t)
    pltpu.sync_copy(sd_hbm, sd_t)
    pltpu.sync_copy(b_hbm, b_v)
    bvec = b_v[...]

    zero16 = jnp.zeros((16,), _f32)

    def _zn(i, _):
        znum[i, :] = zero16
        return _

    lax.fori_loop(0, ZR, _zn, None)

    def _zd(i, _):
        zden[pl.ds(i * 16, 16)] = zero16
        return _

    lax.fori_loop(0, RPT // 16, _zd, None)

    row0 = s * RPT
    for j in range(RPT // ZR):
        pltpu.sync_copy(znum, num_sh.at[pl.ds(row0 + j * ZR, ZR)])
    pltpu.sync_copy(zden, den_sh.at[pl.ds(row0, RPT)])
    plsc.subcore_barrier()

    tile_base = wid * EPT

    def _chunk(ci, _):
        base = tile_base + ci * K
        pltpu.sync_copy(src_hbm.at[pl.ds(base, K)], src_v)
        pltpu.sync_copy(dst_hbm.at[pl.ds(base, K)], dst_v)
        pltpu.async_copy(h_hbm.at[src_v], rows_v, sem).wait()

        for g in range(K // 16):
            s16 = src_v[pl.ds(g * 16, 16)]
            d16 = dst_v[pl.ds(g * 16, 16)]
            e = plsc.load_gather(sa_t, [s16]) + plsc.load_gather(sd_t, [d16])
            e = jnp.where(e > 0, e, 0.2 * e)
            p_v[pl.ds(g * 16, 16)] = jnp.exp(e - bvec)

        def _scale(ei, _):
            pb = plsc.load_gather(p_v, [jnp.full((16,), ei, jnp.int32)])
            rows_v[ei, :] = rows_v[ei, :] * pb
            return _

        lax.fori_loop(0, K, _scale, None, unroll=8)

        pltpu.sync_copy(rows_v, num_sh.at[dst_v], add=True)
        pltpu.sync_copy(p_v, den_sh.at[dst_v], add=True)
        return _

    lax.fori_loop(0, CHUNKS, _chunk, None)

    plsc.subcore_barrier()
    pltpu.sync_copy(num_sh.at[pl.ds(row0, RPT)], num_out.at[c, pl.ds(row0, RPT)])
    pltpu.sync_copy(den_sh.at[pl.ds(row0, RPT)], den_out.at[c, pl.ds(row0, RPT)])


def kernel(x, edge_index, W1, a1_src, a1_dst, b1, W2, a2_src, a2_dst, b2):
    src = edge_index[0].astype(jnp.int32)
    dst = edge_index[1].astype(jnp.int32)
    pad_e = jnp.full((EPAD - E,), N, jnp.int32)
    src_p = jnp.concatenate([src, pad_e])
    dst_p = jnp.concatenate([dst, pad_e])

    x_p = jnp.zeros((NPAD, IN_DIM), _f32).at[:N].set(x)

    h1, sa1, sd1, bnd1 = _dense_in(x_p, W1, a1_src, a1_dst, IN_DIM)
    b16 = jnp.broadcast_to(bnd1.reshape(1), (16,))
    num1, den1 = _sc_edge_pass(src_p, dst_p, h1, sa1.reshape(NPAD),
                               sd1.reshape(NPAD), b16)

    h2, sa2, sd2, bnd2 = _mid(num1, den1.reshape(NC, NPAD, 1), h1, sa1, sd1,
                              bnd1, b1, W2, a2_src, a2_dst)
    b16_2 = jnp.broadcast_to(bnd2.reshape(1), (16,))
    num2, den2 = _sc_edge_pass(src_p, dst_p, h2, sa2.reshape(NPAD),
                               sd2.reshape(NPAD), b16_2)

    out = _final(num2, den2.reshape(NC, NPAD, 1), h2, sa2, sd2, bnd2, b2)
    return out[:N]


# speedup vs baseline: 45.7557x; 45.7557x over previous
import functools

import jax
import jax.numpy as jnp
from jax import lax
from jax.experimental import pallas as pl
from jax.experimental.pallas import tpu as pltpu
from jax.experimental.pallas import tpu_sc as plsc

N = 50000
E = 1600000
IN_DIM = 18
HID = 16

NC = 2
NS = 16
NW = NC * NS
K = 128

NPAD = 50176
RPT = NPAD // NS
ZR = RPT // 8
CHUNKS = -(-E // (NW * K))
EPAD = NW * K * CHUNKS
EPT = EPAD // NW

GB = 16
BR = NPAD // GB

_f32 = jnp.float32


def _accmax(i, ref, val):
    @pl.when(i == 0)
    def _():
        ref[...] = jnp.broadcast_to(val, (1, 1))

    @pl.when(i > 0)
    def _():
        ref[...] = jnp.maximum(ref[...], jnp.broadcast_to(val, (1, 1)))


def _dense_in_body(x_ref, w_ref, asr_ref, adr_ref,
                   h_ref, sa_ref, sd_ref, msa_ref, msd_ref):
    i = pl.program_id(0)
    h = jnp.dot(x_ref[...], w_ref[...], preferred_element_type=_f32)
    sa = jnp.dot(h, asr_ref[...], preferred_element_type=_f32)
    sd = jnp.dot(h, adr_ref[...], preferred_element_type=_f32)
    h_ref[...] = h
    sa_ref[...] = sa
    sd_ref[...] = sd
    _accmax(i, msa_ref, jnp.max(sa))
    _accmax(i, msd_ref, jnp.max(sd))


def _dense_in(x, w, a_src, a_dst, d_in):
    return pl.pallas_call(
        _dense_in_body,
        grid=(GB,),
        in_specs=[
            pl.BlockSpec((BR, d_in), lambda i: (i, 0)),
            pl.BlockSpec((d_in, HID), lambda i: (0, 0)),
            pl.BlockSpec((HID, 1), lambda i: (0, 0)),
            pl.BlockSpec((HID, 1), lambda i: (0, 0)),
        ],
        out_specs=[
            pl.BlockSpec((BR, HID), lambda i: (i, 0)),
            pl.BlockSpec((BR, 1), lambda i: (i, 0)),
            pl.BlockSpec((BR, 1), lambda i: (i, 0)),
            pl.BlockSpec((1, 1), lambda i: (0, 0)),
            pl.BlockSpec((1, 1), lambda i: (0, 0)),
        ],
        out_shape=[
            jax.ShapeDtypeStruct((NPAD, HID), _f32),
            jax.ShapeDtypeStruct((NPAD, 1), _f32),
            jax.ShapeDtypeStruct((NPAD, 1), _f32),
            jax.ShapeDtypeStruct((1, 1), _f32),
            jax.ShapeDtypeStruct((1, 1), _f32),
        ],
    )(x, w.reshape(d_in, HID), a_src.reshape(HID, 1), a_dst.reshape(HID, 1))


def _normalize(num_ref, den_ref, h_ref, sa_ref, sd_ref, b_ref, bias_ref):
    e = sa_ref[...] + sd_ref[...]
    e = jnp.where(e > 0, e, 0.2 * e)
    ps = jnp.exp(e - b_ref[...])
    num = num_ref[0] + num_ref[1] + ps * h_ref[...]
    den = den_ref[0] + den_ref[1] + ps + 1e-16
    return num / den + bias_ref[...]


def _mid_body(num_ref, den_ref, h_ref, sa_ref, sd_ref, b_ref, bias_ref,
              w_ref, asr_ref, adr_ref,
              h2_ref, sa2_ref, sd2_ref, msa_ref, msd_ref):
    i = pl.program_id(0)
    o = _normalize(num_ref, den_ref, h_ref, sa_ref, sd_ref, b_ref, bias_ref)
    o = jnp.where(o > 0, o, jnp.exp(o) - 1.0)
    h2 = jnp.dot(o, w_ref[...], preferred_element_type=_f32)
    sa2 = jnp.dot(h2, asr_ref[...], preferred_element_type=_f32)
    sd2 = jnp.dot(h2, adr_ref[...], preferred_element_type=_f32)
    h2_ref[...] = h2
    sa2_ref[...] = sa2
    sd2_ref[...] = sd2
    _accmax(i, msa_ref, jnp.max(sa2))
    _accmax(i, msd_ref, jnp.max(sd2))


def _mid(num, den, h, sa, sd, b, bias, w, a_src, a_dst):
    return pl.pallas_call(
        _mid_body,
        grid=(GB,),
        in_specs=[
            pl.BlockSpec((NC, BR, HID), lambda i: (0, i, 0)),
            pl.BlockSpec((NC, BR, 1), lambda i: (0, i, 0)),
            pl.BlockSpec((BR, HID), lambda i: (i, 0)),
            pl.BlockSpec((BR, 1), lambda i: (i, 0)),
            pl.BlockSpec((BR, 1), lambda i: (i, 0)),
            pl.BlockSpec((1, 1), lambda i: (0, 0)),
            pl.BlockSpec((1, HID), lambda i: (0, 0)),
            pl.BlockSpec((HID, HID), lambda i: (0, 0)),
            pl.BlockSpec((HID, 1), lambda i: (0, 0)),
            pl.BlockSpec((HID, 1), lambda i: (0, 0)),
        ],
        out_specs=[
            pl.BlockSpec((BR, HID), lambda i: (i, 0)),
            pl.BlockSpec((BR, 1), lambda i: (i, 0)),
            pl.BlockSpec((BR, 1), lambda i: (i, 0)),
            pl.BlockSpec((1, 1), lambda i: (0, 0)),
            pl.BlockSpec((1, 1), lambda i: (0, 0)),
        ],
        out_shape=[
            jax.ShapeDtypeStruct((NPAD, HID), _f32),
            jax.ShapeDtypeStruct((NPAD, 1), _f32),
            jax.ShapeDtypeStruct((NPAD, 1), _f32),
            jax.ShapeDtypeStruct((1, 1), _f32),
            jax.ShapeDtypeStruct((1, 1), _f32),
        ],
    )(num, den, h, sa, sd, b, bias.reshape(1, HID),
      w.reshape(HID, HID), a_src.reshape(HID, 1), a_dst.reshape(HID, 1))


def _final_body(num_ref, den_ref, h_ref, sa_ref, sd_ref, b_ref, bias_ref, o_ref):
    o_ref[...] = _normalize(num_ref, den_ref, h_ref, sa_ref, sd_ref, b_ref,
                            bias_ref)


def _final(num, den, h, sa, sd, b, bias):
    return pl.pallas_call(
        _final_body,
        grid=(GB,),
        in_specs=[
            pl.BlockSpec((NC, BR, HID), lambda i: (0, i, 0)),
            pl.BlockSpec((NC, BR, 1), lambda i: (0, i, 0)),
            pl.BlockSpec((BR, HID), lambda i: (i, 0)),
            pl.BlockSpec((BR, 1), lambda i: (i, 0)),
            pl.BlockSpec((BR, 1), lambda i: (i, 0)),
            pl.BlockSpec((1, 1), lambda i: (0, 0)),
            pl.BlockSpec((1, HID), lambda i: (0, 0)),
        ],
        out_specs=pl.BlockSpec((BR, HID), lambda i: (i, 0)),
        out_shape=jax.ShapeDtypeStruct((NPAD, HID), _f32),
    )(num, den, h, sa, sd, b, bias.reshape(1, HID))


@functools.partial(
    pl.kernel,
    out_type=[
        pltpu.HBM((NC, NPAD, HID), _f32),
        pltpu.HBM((NC * NPAD,), _f32),
    ],
    mesh=plsc.VectorSubcoreMesh(core_axis_name="c", subcore_axis_name="s",
                                num_cores=NC, num_subcores=NS),
    compiler_params=pltpu.CompilerParams(needs_layout_passes=False,
                                         use_tc_tiling_on_sc=False),
    scratch_types=[
        pltpu.VMEM((16,), _f32),
        pltpu.VMEM((K,), jnp.int32),
        pltpu.VMEM((K,), jnp.int32),
        pltpu.VMEM((K, HID), _f32),
        pltpu.VMEM((K,), _f32),
        pltpu.VMEM((K,), _f32),
        pltpu.VMEM((K,), _f32),
        pltpu.VMEM((ZR, HID), _f32),
        pltpu.VMEM((RPT,), _f32),
        pltpu.VMEM_SHARED((NPAD,), _f32),
        pltpu.VMEM_SHARED((NPAD,), _f32),
        pltpu.VMEM_SHARED((NPAD, HID), _f32),
        pltpu.VMEM_SHARED((NPAD,), _f32),
        pltpu.SemaphoreType.DMA,
        pltpu.SemaphoreType.DMA,
    ],
)
def _sc_edge_pass(src_hbm, dst_hbm, h_hbm, sa_hbm, sd_hbm, b_hbm,
                  num_out, den_out,
                  b_v, src_v, dst_v, rows_v, asv, adv, p_v, znum, zden,
                  sa_sh, sd_sh, num_sh, den_sh, sem, sem2):
    c = lax.axis_index("c")
    s = lax.axis_index("s")
    wid = c * NS + s
    row0 = s * RPT

    pltpu.sync_copy(b_hbm, b_v)
    bvec = b_v[...]

    pltpu.sync_copy(sa_hbm.at[pl.ds(row0, RPT)], sa_sh.at[pl.ds(row0, RPT)])
    pltpu.sync_copy(sd_hbm.at[pl.ds(row0, RPT)], sd_sh.at[pl.ds(row0, RPT)])

    zero16 = jnp.zeros((16,), _f32)

    def _zn(i, _):
        znum[i, :] = zero16
        return _

    lax.fori_loop(0, ZR, _zn, None)

    def _zd(i, _):
        zden[pl.ds(i * 16, 16)] = zero16
        return _

    lax.fori_loop(0, RPT // 16, _zd, None)

    for j in range(RPT // ZR):
        pltpu.sync_copy(znum, num_sh.at[pl.ds(row0 + j * ZR, ZR)])
    pltpu.sync_copy(zden, den_sh.at[pl.ds(row0, RPT)])
    plsc.subcore_barrier()

    tile_base = wid * EPT

    def _chunk(ci, _):
        base = tile_base + ci * K
        pltpu.sync_copy(src_hbm.at[pl.ds(base, K)], src_v)
        pltpu.sync_copy(dst_hbm.at[pl.ds(base, K)], dst_v)
        cp_rows = pltpu.async_copy(h_hbm.at[src_v], rows_v, sem)
        cp_as = pltpu.async_copy(sa_sh.at[src_v], asv, sem2)
        cp_ad = pltpu.async_copy(sd_sh.at[dst_v], adv, sem2)
        cp_as.wait()
        cp_ad.wait()

        for g in range(K // 16):
            e = asv[pl.ds(g * 16, 16)] + adv[pl.ds(g * 16, 16)]
            e = jnp.where(e > 0, e, 0.2 * e)
            p_v[pl.ds(g * 16, 16)] = jnp.exp(e - bvec)

        cp_rows.wait()

        def _scale(ei, _):
            pb = plsc.load_gather(p_v, [jnp.full((16,), ei, jnp.int32)])
            rows_v[ei, :] = rows_v[ei, :] * pb
            return _

        lax.fori_loop(0, K, _scale, None, unroll=8)

        pltpu.sync_copy(rows_v, num_sh.at[dst_v], add=True)
        pltpu.sync_copy(p_v, den_sh.at[dst_v], add=True)
        return _

    lax.fori_loop(0, CHUNKS, _chunk, None)

    plsc.subcore_barrier()
    pltpu.sync_copy(num_sh.at[pl.ds(row0, RPT)], num_out.at[c, pl.ds(row0, RPT)])
    pltpu.sync_copy(den_sh.at[pl.ds(row0, RPT)],
                    den_out.at[pl.ds(c * NPAD + row0, RPT)])


def kernel(x, edge_index, W1, a1_src, a1_dst, b1, W2, a2_src, a2_dst, b2):
    src = edge_index[0].astype(jnp.int32)
    dst = edge_index[1].astype(jnp.int32)
    pad_e = jnp.full((EPAD - E,), N, jnp.int32)
    src_p = jnp.concatenate([src, pad_e])
    dst_p = jnp.concatenate([dst, pad_e])

    x_p = jnp.zeros((NPAD, IN_DIM), _f32).at[:N].set(x)

    h1, sa1, sd1, msa1, msd1 = _dense_in(x_p, W1, a1_src, a1_dst, IN_DIM)
    m1 = msa1 + msd1
    bnd1 = jnp.where(m1 > 0, m1, 0.2 * m1)
    b16 = jnp.broadcast_to(bnd1.reshape(1), (16,))
    num1, den1 = _sc_edge_pass(src_p, dst_p, h1, sa1.reshape(NPAD),
                               sd1.reshape(NPAD), b16)

    h2, sa2, sd2, msa2, msd2 = _mid(num1, den1.reshape(NC, NPAD, 1), h1, sa1,
                                    sd1, bnd1, b1, W2, a2_src, a2_dst)
    m2 = msa2 + msd2
    bnd2 = jnp.where(m2 > 0, m2, 0.2 * m2)
    b16_2 = jnp.broadcast_to(bnd2.reshape(1), (16,))
    num2, den2 = _sc_edge_pass(src_p, dst_p, h2, sa2.reshape(NPAD),
                               sd2.reshape(NPAD), b16_2)

    out = _final(num2, den2.reshape(NC, NPAD, 1), h2, sa2, sd2, bnd2, b2)
    return out[:N]


# speedup vs baseline: 87.5442x; 1.9133x over previous
import functools

import jax
import jax.numpy as jnp
from jax import lax
from jax.experimental import pallas as pl
from jax.experimental.pallas import tpu as pltpu
from jax.experimental.pallas import tpu_sc as plsc

N = 50000
E = 1600000
IN_DIM = 18
HID = 16

NC = 2
NS = 16
NW = NC * NS
K = 128

NPAD = 50176
RPT = NPAD // NS
ZR = RPT // 8
CHUNKS = 392
EPAD = NW * K * CHUNKS
EPT = EPAD // NW

GB = 16
BR = NPAD // GB

_f32 = jnp.float32


def _accmax(i, ref, val):
    @pl.when(i == 0)
    def _():
        ref[...] = jnp.broadcast_to(val, (1, 1))

    @pl.when(i > 0)
    def _():
        ref[...] = jnp.maximum(ref[...], jnp.broadcast_to(val, (1, 1)))


def _dense_in_body(x_ref, w_ref, asr_ref, adr_ref,
                   h_ref, sa_ref, sd_ref, msa_ref, msd_ref):
    i = pl.program_id(0)
    h = jnp.dot(x_ref[...], w_ref[...], preferred_element_type=_f32)
    sa = jnp.dot(h, asr_ref[...], preferred_element_type=_f32)
    sd = jnp.dot(h, adr_ref[...], preferred_element_type=_f32)
    h_ref[...] = h
    sa_ref[...] = sa
    sd_ref[...] = sd
    _accmax(i, msa_ref, jnp.max(sa))
    _accmax(i, msd_ref, jnp.max(sd))


def _dense_in(x, w, a_src, a_dst, d_in):
    return pl.pallas_call(
        _dense_in_body,
        grid=(GB,),
        in_specs=[
            pl.BlockSpec((BR, d_in), lambda i: (i, 0)),
            pl.BlockSpec((d_in, HID), lambda i: (0, 0)),
            pl.BlockSpec((HID, 1), lambda i: (0, 0)),
            pl.BlockSpec((HID, 1), lambda i: (0, 0)),
        ],
        out_specs=[
            pl.BlockSpec((BR, HID), lambda i: (i, 0)),
            pl.BlockSpec((BR, 1), lambda i: (i, 0)),
            pl.BlockSpec((BR, 1), lambda i: (i, 0)),
            pl.BlockSpec((1, 1), lambda i: (0, 0)),
            pl.BlockSpec((1, 1), lambda i: (0, 0)),
        ],
        out_shape=[
            jax.ShapeDtypeStruct((NPAD, HID), _f32),
            jax.ShapeDtypeStruct((NPAD, 1), _f32),
            jax.ShapeDtypeStruct((NPAD, 1), _f32),
            jax.ShapeDtypeStruct((1, 1), _f32),
            jax.ShapeDtypeStruct((1, 1), _f32),
        ],
    )(x, w.reshape(d_in, HID), a_src.reshape(HID, 1), a_dst.reshape(HID, 1))


def _normalize(num_ref, den_ref, h_ref, sa_ref, sd_ref, b_ref, bias_ref):
    e = sa_ref[...] + sd_ref[...]
    e = jnp.where(e > 0, e, 0.2 * e)
    ps = jnp.exp(e - b_ref[...])
    num = num_ref[0] + num_ref[1] + ps * h_ref[...]
    den = den_ref[0] + den_ref[1] + ps + 1e-16
    return num / den + bias_ref[...]


def _mid_body(num_ref, den_ref, h_ref, sa_ref, sd_ref, b_ref, bias_ref,
              w_ref, asr_ref, adr_ref,
              h2_ref, sa2_ref, sd2_ref, msa_ref, msd_ref):
    i = pl.program_id(0)
    o = _normalize(num_ref, den_ref, h_ref, sa_ref, sd_ref, b_ref, bias_ref)
    o = jnp.where(o > 0, o, jnp.exp(o) - 1.0)
    h2 = jnp.dot(o, w_ref[...], preferred_element_type=_f32)
    sa2 = jnp.dot(h2, asr_ref[...], preferred_element_type=_f32)
    sd2 = jnp.dot(h2, adr_ref[...], preferred_element_type=_f32)
    h2_ref[...] = h2
    sa2_ref[...] = sa2
    sd2_ref[...] = sd2
    _accmax(i, msa_ref, jnp.max(sa2))
    _accmax(i, msd_ref, jnp.max(sd2))


def _mid(num, den, h, sa, sd, b, bias, w, a_src, a_dst):
    return pl.pallas_call(
        _mid_body,
        grid=(GB,),
        in_specs=[
            pl.BlockSpec((NC, BR, HID), lambda i: (0, i, 0)),
            pl.BlockSpec((NC, BR, 1), lambda i: (0, i, 0)),
            pl.BlockSpec((BR, HID), lambda i: (i, 0)),
            pl.BlockSpec((BR, 1), lambda i: (i, 0)),
            pl.BlockSpec((BR, 1), lambda i: (i, 0)),
            pl.BlockSpec((1, 1), lambda i: (0, 0)),
            pl.BlockSpec((1, HID), lambda i: (0, 0)),
            pl.BlockSpec((HID, HID), lambda i: (0, 0)),
            pl.BlockSpec((HID, 1), lambda i: (0, 0)),
            pl.BlockSpec((HID, 1), lambda i: (0, 0)),
        ],
        out_specs=[
            pl.BlockSpec((BR, HID), lambda i: (i, 0)),
            pl.BlockSpec((BR, 1), lambda i: (i, 0)),
            pl.BlockSpec((BR, 1), lambda i: (i, 0)),
            pl.BlockSpec((1, 1), lambda i: (0, 0)),
            pl.BlockSpec((1, 1), lambda i: (0, 0)),
        ],
        out_shape=[
            jax.ShapeDtypeStruct((NPAD, HID), _f32),
            jax.ShapeDtypeStruct((NPAD, 1), _f32),
            jax.ShapeDtypeStruct((NPAD, 1), _f32),
            jax.ShapeDtypeStruct((1, 1), _f32),
            jax.ShapeDtypeStruct((1, 1), _f32),
        ],
    )(num, den, h, sa, sd, b, bias.reshape(1, HID),
      w.reshape(HID, HID), a_src.reshape(HID, 1), a_dst.reshape(HID, 1))


def _final_body(num_ref, den_ref, h_ref, sa_ref, sd_ref, b_ref, bias_ref, o_ref):
    o_ref[...] = _normalize(num_ref, den_ref, h_ref, sa_ref, sd_ref, b_ref,
                            bias_ref)


def _final(num, den, h, sa, sd, b, bias):
    return pl.pallas_call(
        _final_body,
        grid=(GB,),
        in_specs=[
            pl.BlockSpec((NC, BR, HID), lambda i: (0, i, 0)),
            pl.BlockSpec((NC, BR, 1), lambda i: (0, i, 0)),
            pl.BlockSpec((BR, HID), lambda i: (i, 0)),
            pl.BlockSpec((BR, 1), lambda i: (i, 0)),
            pl.BlockSpec((BR, 1), lambda i: (i, 0)),
            pl.BlockSpec((1, 1), lambda i: (0, 0)),
            pl.BlockSpec((1, HID), lambda i: (0, 0)),
        ],
        out_specs=pl.BlockSpec((BR, HID), lambda i: (i, 0)),
        out_shape=jax.ShapeDtypeStruct((NPAD, HID), _f32),
    )(num, den, h, sa, sd, b, bias.reshape(1, HID))


@functools.partial(
    pl.kernel,
    out_type=[
        pltpu.HBM((NC, NPAD, HID), _f32),
        pltpu.HBM((NC * NPAD,), _f32),
    ],
    mesh=plsc.VectorSubcoreMesh(core_axis_name="c", subcore_axis_name="s",
                                num_cores=NC, num_subcores=NS),
    compiler_params=pltpu.CompilerParams(needs_layout_passes=False,
                                         use_tc_tiling_on_sc=False),
    scratch_types=[
        pltpu.VMEM((16,), _f32),
        [pltpu.VMEM((K,), jnp.int32) for _ in range(4)],
        [pltpu.VMEM((K,), jnp.int32) for _ in range(4)],
        [pltpu.VMEM((K, HID), _f32) for _ in range(2)],
        [pltpu.VMEM((K,), _f32) for _ in range(2)],
        [pltpu.VMEM((K,), _f32) for _ in range(2)],
        pltpu.VMEM((K,), _f32),
        pltpu.VMEM((ZR, HID), _f32),
        pltpu.VMEM((RPT,), _f32),
        pltpu.VMEM_SHARED((NPAD,), _f32),
        pltpu.VMEM_SHARED((NPAD,), _f32),
        pltpu.VMEM_SHARED((NPAD, HID), _f32),
        pltpu.VMEM_SHARED((NPAD,), _f32),
        [pltpu.SemaphoreType.DMA for _ in range(4)],
        pltpu.SemaphoreType.DMA,
        pltpu.SemaphoreType.DMA,
        pltpu.SemaphoreType.DMA,
    ],
)
def _sc_edge_pass(src_hbm, dst_hbm, h_hbm, sa_hbm, sd_hbm, b_hbm,
                  num_out, den_out,
                  b_v, src_v, dst_v, rows_v, asv, adv, p_v, znum, zden,
                  sa_sh, sd_sh, num_sh, den_sh, isem, gsem, asem, dsem):
    c = lax.axis_index("c")
    s = lax.axis_index("s")
    wid = c * NS + s
    row0 = s * RPT

    pltpu.sync_copy(b_hbm, b_v)
    bvec = b_v[...]

    pltpu.sync_copy(sa_hbm.at[pl.ds(row0, RPT)], sa_sh.at[pl.ds(row0, RPT)])
    pltpu.sync_copy(sd_hbm.at[pl.ds(row0, RPT)], sd_sh.at[pl.ds(row0, RPT)])

    zero16 = jnp.zeros((16,), _f32)

    def _zn(i, _):
        znum[i, :] = zero16
        return _

    lax.fori_loop(0, ZR, _zn, None)

    def _zd(i, _):
        zden[pl.ds(i * 16, 16)] = zero16
        return _

    lax.fori_loop(0, RPT // 16, _zd, None)

    for j in range(RPT // ZR):
        pltpu.sync_copy(znum, num_sh.at[pl.ds(row0 + j * ZR, ZR)])
    pltpu.sync_copy(zden, den_sh.at[pl.ds(row0, RPT)])
    plsc.subcore_barrier()

    tile_base = wid * EPT

    def _issue_idx(ci, slot):
        base = tile_base + ci * K
        pltpu.async_copy(src_hbm.at[pl.ds(base, K)], src_v[slot], isem[slot])
        pltpu.async_copy(dst_hbm.at[pl.ds(base, K)], dst_v[slot], isem[slot])

    def _wait_idx(slot):
        pltpu.make_async_copy(src_hbm.at[pl.ds(0, K)], src_v[slot],
                              isem[slot]).wait()
        pltpu.make_async_copy(dst_hbm.at[pl.ds(0, K)], dst_v[slot],
                              isem[slot]).wait()

    def _issue_gathers(slot, par):
        pltpu.async_copy(h_hbm.at[src_v[slot]], rows_v[par], gsem)
        pltpu.async_copy(sa_sh.at[src_v[slot]], asv[par], asem)
        pltpu.async_copy(sd_sh.at[dst_v[slot]], adv[par], dsem)

    def _wait_gathers(slot, par):
        pltpu.make_async_copy(h_hbm.at[src_v[slot]], rows_v[par], gsem).wait()
        pltpu.make_async_copy(sa_sh.at[src_v[slot]], asv[par], asem).wait()
        pltpu.make_async_copy(sd_sh.at[dst_v[slot]], adv[par], dsem).wait()

    _issue_idx(0, 0)
    _issue_idx(1, 1)
    _wait_idx(0)
    _issue_gathers(0, 0)

    def _outer(io, _):
        for b in range(4):
            ci = io * 4 + b
            par = b % 2
            _wait_gathers(b, par)

            @pl.when(ci < CHUNKS - 1)
            def _():
                _wait_idx((b + 1) % 4)
                _issue_gathers((b + 1) % 4, (b + 1) % 2)

            @pl.when(ci < CHUNKS - 2)
            def _():
                _issue_idx(ci + 2, (b + 2) % 4)

            for g in range(K // 16):
                e = (asv[par][pl.ds(g * 16, 16)]
                     + adv[par][pl.ds(g * 16, 16)])
                e = jnp.where(e > 0, e, 0.2 * e)
                p_v[pl.ds(g * 16, 16)] = jnp.exp(e - bvec)

            rv = rows_v[par]

            def _scale(ei, _):
                pb = plsc.load_gather(p_v, [jnp.full((16,), ei, jnp.int32)])
                rv[ei, :] = rv[ei, :] * pb
                return _

            lax.fori_loop(0, K, _scale, None, unroll=8)

            pltpu.sync_copy(rows_v[par], num_sh.at[dst_v[b]], add=True)
            pltpu.sync_copy(p_v, den_sh.at[dst_v[b]], add=True)
        return _

    lax.fori_loop(0, CHUNKS // 4, _outer, None)

    plsc.subcore_barrier()
    pltpu.sync_copy(num_sh.at[pl.ds(row0, RPT)], num_out.at[c, pl.ds(row0, RPT)])
    pltpu.sync_copy(den_sh.at[pl.ds(row0, RPT)],
                    den_out.at[pl.ds(c * NPAD + row0, RPT)])


def kernel(x, edge_index, W1, a1_src, a1_dst, b1, W2, a2_src, a2_dst, b2):
    src = edge_index[0].astype(jnp.int32)
    dst = edge_index[1].astype(jnp.int32)
    pad_e = jnp.full((EPAD - E,), N, jnp.int32)
    src_p = jnp.concatenate([src, pad_e])
    dst_p = jnp.concatenate([dst, pad_e])

    x_p = jnp.zeros((NPAD, IN_DIM), _f32).at[:N].set(x)

    h1, sa1, sd1, msa1, msd1 = _dense_in(x_p, W1, a1_src, a1_dst, IN_DIM)
    m1 = msa1 + msd1
    bnd1 = jnp.where(m1 > 0, m1, 0.2 * m1)
    b16 = jnp.broadcast_to(bnd1.reshape(1), (16,))
    num1, den1 = _sc_edge_pass(src_p, dst_p, h1, sa1.reshape(NPAD),
                               sd1.reshape(NPAD), b16)

    h2, sa2, sd2, msa2, msd2 = _mid(num1, den1.reshape(NC, NPAD, 1), h1, sa1,
                                    sd1, bnd1, b1, W2, a2_src, a2_dst)
    m2 = msa2 + msd2
    bnd2 = jnp.where(m2 > 0, m2, 0.2 * m2)
    b16_2 = jnp.broadcast_to(bnd2.reshape(1), (16,))
    num2, den2 = _sc_edge_pass(src_p, dst_p, h2, sa2.reshape(NPAD),
                               sd2.reshape(NPAD), b16_2)

    out = _final(num2, den2.reshape(NC, NPAD, 1), h2, sa2, sd2, bnd2, b2)
    return out[:N]


# speedup vs baseline: 97.6598x; 1.1155x over previous
import functools

import jax
import jax.numpy as jnp
from jax import lax
from jax.experimental import pallas as pl
from jax.experimental.pallas import tpu as pltpu
from jax.experimental.pallas import tpu_sc as plsc

N = 50000
E = 1600000
IN_DIM = 18
HID = 16

NC = 2
NS = 16
NW = NC * NS
K = 128

NPAD = 50176
RPT = NPAD // NS
ZR = RPT // 8
CHUNKS = 392
EPAD = NW * K * CHUNKS
EPT = EPAD // NW

GB = 16
BR = NPAD // GB

_f32 = jnp.float32


def _accmax(i, ref, val):
    @pl.when(i == 0)
    def _():
        ref[...] = jnp.broadcast_to(val, (1, 1))

    @pl.when(i > 0)
    def _():
        ref[...] = jnp.maximum(ref[...], jnp.broadcast_to(val, (1, 1)))


def _dense_in_body(x_ref, w_ref, asr_ref, adr_ref,
                   h_ref, sa_ref, sd_ref, msa_ref, msd_ref):
    i = pl.program_id(0)
    h = jnp.dot(x_ref[...], w_ref[...], preferred_element_type=_f32)
    sa = jnp.dot(h, asr_ref[...], preferred_element_type=_f32)
    sd = jnp.dot(h, adr_ref[...], preferred_element_type=_f32)
    h_ref[...] = h
    sa_ref[...] = sa
    sd_ref[...] = sd
    _accmax(i, msa_ref, jnp.max(sa))
    _accmax(i, msd_ref, jnp.max(sd))


def _dense_in(x, w, a_src, a_dst, d_in):
    return pl.pallas_call(
        _dense_in_body,
        grid=(GB,),
        in_specs=[
            pl.BlockSpec((BR, d_in), lambda i: (i, 0)),
            pl.BlockSpec((d_in, HID), lambda i: (0, 0)),
            pl.BlockSpec((HID, 1), lambda i: (0, 0)),
            pl.BlockSpec((HID, 1), lambda i: (0, 0)),
        ],
        out_specs=[
            pl.BlockSpec((BR, HID), lambda i: (i, 0)),
            pl.BlockSpec((BR, 1), lambda i: (i, 0)),
            pl.BlockSpec((BR, 1), lambda i: (i, 0)),
            pl.BlockSpec((1, 1), lambda i: (0, 0)),
            pl.BlockSpec((1, 1), lambda i: (0, 0)),
        ],
        out_shape=[
            jax.ShapeDtypeStruct((NPAD, HID), _f32),
            jax.ShapeDtypeStruct((NPAD, 1), _f32),
            jax.ShapeDtypeStruct((NPAD, 1), _f32),
            jax.ShapeDtypeStruct((1, 1), _f32),
            jax.ShapeDtypeStruct((1, 1), _f32),
        ],
    )(x, w.reshape(d_in, HID), a_src.reshape(HID, 1), a_dst.reshape(HID, 1))


def _normalize(num_ref, den_ref, h_ref, sa_ref, sd_ref, b_ref, bias_ref):
    e = sa_ref[...] + sd_ref[...]
    e = jnp.where(e > 0, e, 0.2 * e)
    ps = jnp.exp(e - b_ref[...])
    num = num_ref[0] + num_ref[1] + ps * h_ref[...]
    den = den_ref[0] + den_ref[1] + ps + 1e-16
    return num / den + bias_ref[...]


def _mid_body(num_ref, den_ref, h_ref, sa_ref, sd_ref, b_ref, bias_ref,
              w_ref, asr_ref, adr_ref,
              h2_ref, sa2_ref, sd2_ref, msa_ref, msd_ref):
    i = pl.program_id(0)
    o = _normalize(num_ref, den_ref, h_ref, sa_ref, sd_ref, b_ref, bias_ref)
    o = jnp.where(o > 0, o, jnp.exp(o) - 1.0)
    h2 = jnp.dot(o, w_ref[...], preferred_element_type=_f32)
    sa2 = jnp.dot(h2, asr_ref[...], preferred_element_type=_f32)
    sd2 = jnp.dot(h2, adr_ref[...], preferred_element_type=_f32)
    h2_ref[...] = h2
    sa2_ref[...] = sa2
    sd2_ref[...] = sd2
    _accmax(i, msa_ref, jnp.max(sa2))
    _accmax(i, msd_ref, jnp.max(sd2))


def _mid(num, den, h, sa, sd, b, bias, w, a_src, a_dst):
    return pl.pallas_call(
        _mid_body,
        grid=(GB,),
        in_specs=[
            pl.BlockSpec((NC, BR, HID), lambda i: (0, i, 0)),
            pl.BlockSpec((NC, BR, 1), lambda i: (0, i, 0)),
            pl.BlockSpec((BR, HID), lambda i: (i, 0)),
            pl.BlockSpec((BR, 1), lambda i: (i, 0)),
            pl.BlockSpec((BR, 1), lambda i: (i, 0)),
            pl.BlockSpec((1, 1), lambda i: (0, 0)),
            pl.BlockSpec((1, HID), lambda i: (0, 0)),
            pl.BlockSpec((HID, HID), lambda i: (0, 0)),
            pl.BlockSpec((HID, 1), lambda i: (0, 0)),
            pl.BlockSpec((HID, 1), lambda i: (0, 0)),
        ],
        out_specs=[
            pl.BlockSpec((BR, HID), lambda i: (i, 0)),
            pl.BlockSpec((BR, 1), lambda i: (i, 0)),
            pl.BlockSpec((BR, 1), lambda i: (i, 0)),
            pl.BlockSpec((1, 1), lambda i: (0, 0)),
            pl.BlockSpec((1, 1), lambda i: (0, 0)),
        ],
        out_shape=[
            jax.ShapeDtypeStruct((NPAD, HID), _f32),
            jax.ShapeDtypeStruct((NPAD, 1), _f32),
            jax.ShapeDtypeStruct((NPAD, 1), _f32),
            jax.ShapeDtypeStruct((1, 1), _f32),
            jax.ShapeDtypeStruct((1, 1), _f32),
        ],
    )(num, den, h, sa, sd, b, bias.reshape(1, HID),
      w.reshape(HID, HID), a_src.reshape(HID, 1), a_dst.reshape(HID, 1))


def _final_body(num_ref, den_ref, h_ref, sa_ref, sd_ref, b_ref, bias_ref, o_ref):
    o_ref[...] = _normalize(num_ref, den_ref, h_ref, sa_ref, sd_ref, b_ref,
                            bias_ref)


def _final(num, den, h, sa, sd, b, bias):
    return pl.pallas_call(
        _final_body,
        grid=(GB,),
        in_specs=[
            pl.BlockSpec((NC, BR, HID), lambda i: (0, i, 0)),
            pl.BlockSpec((NC, BR, 1), lambda i: (0, i, 0)),
            pl.BlockSpec((BR, HID), lambda i: (i, 0)),
            pl.BlockSpec((BR, 1), lambda i: (i, 0)),
            pl.BlockSpec((BR, 1), lambda i: (i, 0)),
            pl.BlockSpec((1, 1), lambda i: (0, 0)),
            pl.BlockSpec((1, HID), lambda i: (0, 0)),
        ],
        out_specs=pl.BlockSpec((BR, HID), lambda i: (i, 0)),
        out_shape=jax.ShapeDtypeStruct((NPAD, HID), _f32),
    )(num, den, h, sa, sd, b, bias.reshape(1, HID))


@functools.partial(
    pl.kernel,
    out_type=[
        pltpu.HBM((NC, NPAD, HID), _f32),
        pltpu.HBM((NC * NPAD,), _f32),
    ],
    mesh=plsc.VectorSubcoreMesh(core_axis_name="c", subcore_axis_name="s",
                                num_cores=NC, num_subcores=NS),
    compiler_params=pltpu.CompilerParams(needs_layout_passes=False,
                                         use_tc_tiling_on_sc=False),
    scratch_types=[
        pltpu.VMEM((16,), _f32),
        [pltpu.VMEM((K,), jnp.int32) for _ in range(4)],
        [pltpu.VMEM((K,), jnp.int32) for _ in range(4)],
        [pltpu.VMEM((K, HID), _f32) for _ in range(2)],
        [pltpu.VMEM((K,), _f32) for _ in range(2)],
        [pltpu.VMEM((K,), _f32) for _ in range(2)],
        [pltpu.VMEM((K,), _f32) for _ in range(2)],
        pltpu.VMEM((ZR, HID), _f32),
        pltpu.VMEM((RPT,), _f32),
        pltpu.VMEM_SHARED((NPAD,), _f32),
        pltpu.VMEM_SHARED((NPAD,), _f32),
        pltpu.VMEM_SHARED((NPAD, HID), _f32),
        pltpu.VMEM_SHARED((NPAD,), _f32),
        [pltpu.SemaphoreType.DMA for _ in range(4)],
        pltpu.SemaphoreType.DMA,
        pltpu.SemaphoreType.DMA,
        pltpu.SemaphoreType.DMA,
    ],
)
def _sc_edge_pass(src_hbm, dst_hbm, h_hbm, sa_hbm, sd_hbm, b_hbm,
                  num_out, den_out,
                  b_v, src_v, dst_v, rows_v, asv, adv, p_v, znum, zden,
                  sa_sh, sd_sh, num_sh, den_sh, isem, gsem, asem, dsem):
    c = lax.axis_index("c")
    s = lax.axis_index("s")
    wid = c * NS + s
    row0 = s * RPT

    pltpu.sync_copy(b_hbm, b_v)
    bvec = b_v[...]

    pltpu.sync_copy(sa_hbm.at[pl.ds(row0, RPT)], sa_sh.at[pl.ds(row0, RPT)])
    pltpu.sync_copy(sd_hbm.at[pl.ds(row0, RPT)], sd_sh.at[pl.ds(row0, RPT)])

    zero16 = jnp.zeros((16,), _f32)

    def _zn(i, _):
        znum[i, :] = zero16
        return _

    lax.fori_loop(0, ZR, _zn, None)

    def _zd(i, _):
        zden[pl.ds(i * 16, 16)] = zero16
        return _

    lax.fori_loop(0, RPT // 16, _zd, None)

    for j in range(RPT // ZR):
        pltpu.sync_copy(znum, num_sh.at[pl.ds(row0 + j * ZR, ZR)])
    pltpu.sync_copy(zden, den_sh.at[pl.ds(row0, RPT)])
    plsc.subcore_barrier()

    tile_base = wid * EPT

    def _issue_idx(ci, slot):
        base = tile_base + ci * K
        pltpu.async_copy(src_hbm.at[pl.ds(base, K)], src_v[slot], isem[slot])
        pltpu.async_copy(dst_hbm.at[pl.ds(base, K)], dst_v[slot], isem[slot])

    def _wait_idx(slot):
        pltpu.make_async_copy(src_hbm.at[pl.ds(0, K)], src_v[slot],
                              isem[slot]).wait()
        pltpu.make_async_copy(dst_hbm.at[pl.ds(0, K)], dst_v[slot],
                              isem[slot]).wait()

    def _issue_gathers(slot, par):
        pltpu.async_copy(h_hbm.at[src_v[slot]], rows_v[par], gsem)
        pltpu.async_copy(sa_sh.at[src_v[slot]], asv[par], asem)
        pltpu.async_copy(sd_sh.at[dst_v[slot]], adv[par], dsem)

    def _wait_gathers(slot, par):
        pltpu.make_async_copy(h_hbm.at[src_v[slot]], rows_v[par], gsem).wait()
        pltpu.make_async_copy(sa_sh.at[src_v[slot]], asv[par], asem).wait()
        pltpu.make_async_copy(sd_sh.at[dst_v[slot]], adv[par], dsem).wait()

    _issue_idx(0, 0)
    _issue_idx(1, 1)
    _wait_idx(0)
    _issue_gathers(0, 0)

    def _outer(io, _):
        for b in range(4):
            ci = io * 4 + b
            par = b % 2
            opar = (b + 1) % 2
            _wait_gathers(b, par)

            @pl.when(ci < CHUNKS - 1)
            def _():
                _wait_idx((b + 1) % 4)
                _issue_gathers((b + 1) % 4, opar)

            @pl.when(ci < CHUNKS - 2)
            def _():
                _issue_idx(ci + 2, (b + 2) % 4)

            rv = rows_v[par]
            pv = p_v[par]
            for g in range(K // 16):
                e = (asv[par][pl.ds(g * 16, 16)]
                     + adv[par][pl.ds(g * 16, 16)])
                e = jnp.where(e > 0, e, 0.2 * e)
                p16 = jnp.exp(e - bvec)
                pv[pl.ds(g * 16, 16)] = p16
                for l in range(16):
                    ei = g * 16 + l
                    pb = lax.gather(
                        p16, jnp.full((16, 1), l, jnp.int32),
                        lax.GatherDimensionNumbers(
                            offset_dims=(), collapsed_slice_dims=(0,),
                            start_index_map=(0,)),
                        slice_sizes=(1,),
                        mode=lax.GatherScatterMode.PROMISE_IN_BOUNDS)
                    rv[ei, :] = rv[ei, :] * pb

            pltpu.sync_copy(rows_v[par], num_sh.at[dst_v[b]], add=True)
            pltpu.sync_copy(pv, den_sh.at[dst_v[b]], add=True)
        return _

    lax.fori_loop(0, CHUNKS // 4, _outer, None)

    plsc.subcore_barrier()
    pltpu.sync_copy(num_sh.at[pl.ds(row0, RPT)], num_out.at[c, pl.ds(row0, RPT)])
    pltpu.sync_copy(den_sh.at[pl.ds(row0, RPT)],
                    den_out.at[pl.ds(c * NPAD + row0, RPT)])


def kernel(x, edge_index, W1, a1_src, a1_dst, b1, W2, a2_src, a2_dst, b2):
    src = edge_index[0].astype(jnp.int32)
    dst = edge_index[1].astype(jnp.int32)
    pad_e = jnp.full((EPAD - E,), N, jnp.int32)
    src_p = jnp.concatenate([src, pad_e])
    dst_p = jnp.concatenate([dst, pad_e])

    x_p = jnp.zeros((NPAD, IN_DIM), _f32).at[:N].set(x)

    h1, sa1, sd1, msa1, msd1 = _dense_in(x_p, W1, a1_src, a1_dst, IN_DIM)
    m1 = msa1 + msd1
    bnd1 = jnp.where(m1 > 0, m1, 0.2 * m1)
    b16 = jnp.broadcast_to(bnd1.reshape(1), (16,))
    num1, den1 = _sc_edge_pass(src_p, dst_p, h1, sa1.reshape(NPAD),
                               sd1.reshape(NPAD), b16)

    h2, sa2, sd2, msa2, msd2 = _mid(num1, den1.reshape(NC, NPAD, 1), h1, sa1,
                                    sd1, bnd1, b1, W2, a2_src, a2_dst)
    m2 = msa2 + msd2
    bnd2 = jnp.where(m2 > 0, m2, 0.2 * m2)
    b16_2 = jnp.broadcast_to(bnd2.reshape(1), (16,))
    num2, den2 = _sc_edge_pass(src_p, dst_p, h2, sa2.reshape(NPAD),
                               sd2.reshape(NPAD), b16_2)

    out = _final(num2, den2.reshape(NC, NPAD, 1), h2, sa2, sd2, bnd2, b2)
    return out[:N]
